# baseline (speedup 1.0000x reference)
import functools

import jax
import jax.numpy as jnp
from jax import lax
from jax.experimental import pallas as pl
from jax.experimental.pallas import tpu as pltpu
from jax.experimental.pallas import tpu_sc as plsc

NC = 2
NS = 16
NW = NC * NS
E_CHUNK = 80
NBUF = 4
_GDN = lax.GatherDimensionNumbers(
    offset_dims=(), collapsed_slice_dims=(0,), start_index_map=(0,))


def _vgather16(v, idx):
    return lax.gather(v, idx[:, None], _GDN, (1,),
                      mode=lax.GatherScatterMode.PROMISE_IN_BOUNDS)


def _segment_sum_sc(h, nbrs_flat, zeros, n_pad):
    n_edges, d = h.shape
    edges_per_w = n_edges // NW
    nchunk = edges_per_w // E_CHUNK
    rows_per_s = n_pad // NS

    mesh = plsc.VectorSubcoreMesh(core_axis_name="c", subcore_axis_name="s")

    @functools.partial(
        pl.kernel,
        mesh=mesh,
        out_type=jax.ShapeDtypeStruct((NC, n_pad, d), jnp.float32),
        scratch_types=(
            [pltpu.VMEM((E_CHUNK, d), jnp.float32)] * NBUF
            + [pltpu.VMEM((2 * E_CHUNK,), jnp.int32)] * NBUF
            + [pltpu.VMEM((E_CHUNK,), jnp.int32)] * NBUF
            + [pltpu.VMEM_SHARED((n_pad, d), jnp.float32)]
            + [pltpu.SemaphoreType.DMA] * NBUF
        ),
    )
    def segsum(h_hbm, nbrs_hbm, z_hbm, out_hbm, *rest):
        rows = rest[:NBUF]
        pairb = rest[NBUF:2 * NBUF]
        idxb = rest[2 * NBUF:3 * NBUF]
        acc_sh = rest[3 * NBUF]
        lsem = rest[3 * NBUF + 1:]
        c = lax.axis_index("c")
        s = lax.axis_index("s")
        wid = s * NC + c
        srow = s * rows_per_s
        for j in range(rows_per_s // E_CHUNK):
            pltpu.sync_copy(z_hbm, acc_sh.at[pl.ds(srow + j * E_CHUNK, E_CHUNK)])
        if rows_per_s % E_CHUNK:
            pltpu.sync_copy(
                z_hbm.at[pl.ds(0, rows_per_s % E_CHUNK), :],
                acc_sh.at[pl.ds(srow + (rows_per_s // E_CHUNK) * E_CHUNK,
                                rows_per_s % E_CHUNK)])
        plsc.subcore_barrier()
        iota16 = lax.iota(jnp.int32, 16)
        idx_ev = (iota16 % 8) * 2
        base = wid * edges_per_w

        def start_load(i, b):
            blk = jnp.minimum(i, nchunk - 1)
            off = base + blk * E_CHUNK
            pltpu.make_async_copy(
                nbrs_hbm.at[pl.ds(2 * off, 2 * E_CHUNK)], pairb[b],
                lsem[b]).start()
            pltpu.make_async_copy(
                h_hbm.at[pl.ds(off, E_CHUNK), :], rows[b], lsem[b]).start()

        for b in range(NBUF):
            start_load(jnp.int32(b), b)

        def body(i, carry):
            for b in range(NBUF):
                @pl.when(i % NBUF == b)
                def _():
                    pltpu.make_async_copy(
                        nbrs_hbm.at[pl.ds(0, 2 * E_CHUNK)], pairb[b],
                        lsem[b]).wait()
                    pltpu.make_async_copy(
                        h_hbm.at[pl.ds(0, E_CHUNK), :], rows[b],
                        lsem[b]).wait()
                    for k in range(E_CHUNK // 16):
                        v0 = pairb[b][pl.ds(32 * k, 16)]
                        v1 = pairb[b][pl.ds(32 * k + 16, 16)]
                        lo = _vgather16(v0, idx_ev)
                        hi = _vgather16(v1, idx_ev)
                        idxb[b][pl.ds(k * 16, 16)] = jnp.where(
                            iota16 < 8, lo, hi)
                    pltpu.sync_copy(rows[b], acc_sh.at[idxb[b]], add=True)
                    start_load(i + NBUF, b)
            return carry

        lax.fori_loop(0, nchunk, body, 0)
        for b in range(NBUF):
            pltpu.make_async_copy(
                nbrs_hbm.at[pl.ds(0, 2 * E_CHUNK)], pairb[b], lsem[b]).wait()
            pltpu.make_async_copy(
                h_hbm.at[pl.ds(0, E_CHUNK), :], rows[b], lsem[b]).wait()
        plsc.subcore_barrier()
        pltpu.sync_copy(acc_sh.at[pl.ds(s * rows_per_s, rows_per_s)],
                        out_hbm.at[c, pl.ds(s * rows_per_s, rows_per_s)])

    return segsum(h, nbrs_flat, zeros)


def _mlp_tc(r, partials, W):
    n, d = r.shape
    blk = 2000

    def body(r_ref, p_ref, w_ref, o_ref):
        w = w_ref[...]
        msg = p_ref[0] + p_ref[1]
        acc = jnp.dot(r_ref[...], w[:d], preferred_element_type=jnp.float32)
        acc = acc + jnp.dot(msg, w[d:], preferred_element_type=jnp.float32)
        o_ref[...] = jnp.maximum(acc, 0.0)

    return pl.pallas_call(
        body,
        grid=(n // blk,),
        in_specs=[
            pl.BlockSpec((blk, d), lambda i: (i, 0)),
            pl.BlockSpec((2, blk, d), lambda i: (0, i, 0)),
            pl.BlockSpec((2 * d, d), lambda i: (0, 0)),
        ],
        out_specs=pl.BlockSpec((blk, d), lambda i: (i, 0)),
        out_shape=jax.ShapeDtypeStruct((n, d), jnp.float32),
    )(r, partials, W)


def kernel(r, h, nbrs, W):
    n_nodes, d = r.shape
    n_pad = ((n_nodes + NS * 8 - 1) // (NS * 8)) * (NS * 8)
    zeros = jnp.zeros((E_CHUNK, d), jnp.float32)
    nbrs_flat = nbrs.astype(jnp.int32).reshape(-1)
    partials = _segment_sum_sc(h, nbrs_flat, zeros, n_pad)
    return _mlp_tc(r, partials, W)


# speedup vs baseline: 2.1453x; 2.1453x over previous
import functools

import jax
import jax.numpy as jnp
from jax import lax
from jax.experimental import pallas as pl
from jax.experimental.pallas import tpu as pltpu
from jax.experimental.pallas import tpu_sc as plsc

NC = 2
NS = 16
NW = NC * NS
E_CHUNK = 80
NBUF = 3


def _segment_sum_sc(h, dst3, zeros, n_pad):
    n_edges, d = h.shape
    edges_per_w = n_edges // NW
    nchunk = edges_per_w // E_CHUNK
    rows_per_s = n_pad // NS

    mesh = plsc.VectorSubcoreMesh(core_axis_name="c", subcore_axis_name="s")

    @functools.partial(
        pl.kernel,
        mesh=mesh,
        out_type=jax.ShapeDtypeStruct((NC, n_pad, d), jnp.float32),
        scratch_types=[
            pltpu.VMEM((nchunk, E_CHUNK), jnp.int32),
        ] + [pltpu.VMEM((E_CHUNK, d), jnp.float32)] * NBUF + [
            pltpu.VMEM_SHARED((n_pad, d), jnp.float32),
        ] + [pltpu.SemaphoreType.DMA] * (NBUF + 1),
    )
    def segsum(h_hbm, dst_hbm, z_hbm, out_hbm, idx_v, *rest):
        rows = rest[:NBUF]
        acc_sh = rest[NBUF]
        lsem = rest[NBUF + 1:NBUF + 1 + NBUF]
        zsem = rest[2 * NBUF + 1]
        c = lax.axis_index("c")
        s = lax.axis_index("s")
        wid = s * NC + c
        srow = s * rows_per_s
        nz_full, z_tail = divmod(rows_per_s, E_CHUNK)
        zcopies = [
            pltpu.make_async_copy(
                z_hbm, acc_sh.at[pl.ds(srow + j * E_CHUNK, E_CHUNK)], zsem)
            for j in range(nz_full)]
        if z_tail:
            zcopies.append(pltpu.make_async_copy(
                z_hbm.at[pl.ds(0, z_tail), :],
                acc_sh.at[pl.ds(srow + nz_full * E_CHUNK, z_tail)], zsem))
        for cp in zcopies:
            cp.start()
        pltpu.sync_copy(dst_hbm.at[wid], idx_v)
        base = wid * edges_per_w

        def start_load(i, b):
            blk = jnp.minimum(i, nchunk - 1)
            pltpu.make_async_copy(
                h_hbm.at[pl.ds(base + blk * E_CHUNK, E_CHUNK), :], rows[b],
                lsem[b]).start()

        for b in range(NBUF):
            start_load(jnp.int32(b), b)

        for cp in zcopies:
            cp.wait()
        plsc.subcore_barrier()

        def body(i, carry):
            for b in range(NBUF):
                @pl.when(i % NBUF == b)
                def _():
                    pltpu.make_async_copy(
                        h_hbm.at[pl.ds(0, E_CHUNK), :], rows[b],
                        lsem[b]).wait()
                    pltpu.sync_copy(rows[b], acc_sh.at[idx_v.at[i]], add=True)
                    start_load(i + NBUF, b)
            return carry

        lax.fori_loop(0, nchunk, body, 0)
        for b in range(NBUF):
            pltpu.make_async_copy(
                h_hbm.at[pl.ds(0, E_CHUNK), :], rows[b], lsem[b]).wait()
        plsc.subcore_barrier()
        pltpu.sync_copy(acc_sh.at[pl.ds(srow, rows_per_s)],
                        out_hbm.at[c, pl.ds(srow, rows_per_s)])

    return segsum(h, dst3, zeros)


def _mlp_tc(r, partials, W):
    n, d = r.shape
    blk = 2000

    def body(r_ref, p_ref, w_ref, o_ref):
        w = w_ref[...]
        msg = p_ref[0] + p_ref[1]
        acc = jnp.dot(r_ref[...], w[:d], preferred_element_type=jnp.float32)
        acc = acc + jnp.dot(msg, w[d:], preferred_element_type=jnp.float32)
        o_ref[...] = jnp.maximum(acc, 0.0)

    return pl.pallas_call(
        body,
        grid=(n // blk,),
        in_specs=[
            pl.BlockSpec((blk, d), lambda i: (i, 0)),
            pl.BlockSpec((2, blk, d), lambda i: (0, i, 0)),
            pl.BlockSpec((2 * d, d), lambda i: (0, 0)),
        ],
        out_specs=pl.BlockSpec((blk, d), lambda i: (i, 0)),
        out_shape=jax.ShapeDtypeStruct((n, d), jnp.float32),
    )(r, partials, W)


def kernel(r, h, nbrs, W):
    n_nodes, d = r.shape
    n_pad = ((n_nodes + NS * 8 - 1) // (NS * 8)) * (NS * 8)
    n_edges = h.shape[0]
    edges_per_w = n_edges // NW
    dst3 = nbrs[:, 0].astype(jnp.int32).reshape(
        NW, edges_per_w // E_CHUNK, E_CHUNK)
    zeros = jnp.zeros((E_CHUNK, d), jnp.float32)
    partials = _segment_sum_sc(h, dst3, zeros, n_pad)
    return _mlp_tc(r, partials, W)


# speedup vs baseline: 2.3546x; 1.0976x over previous
import functools

import jax
import jax.numpy as jnp
from jax import lax
from jax.experimental import pallas as pl
from jax.experimental.pallas import tpu as pltpu
from jax.experimental.pallas import tpu_sc as plsc

NC = 2
NS = 16
NW = NC * NS
E_CHUNK = 80
NBUF = 3


def _segment_sum_sc(h, dst3, zeros, n_pad):
    n_edges, d = h.shape
    edges_per_w = n_edges // NW
    nchunk = edges_per_w // E_CHUNK
    rows_per_s = n_pad // NS

    mesh = plsc.VectorSubcoreMesh(core_axis_name="c", subcore_axis_name="s")

    @functools.partial(
        pl.kernel,
        mesh=mesh,
        out_type=jax.ShapeDtypeStruct((NC, n_pad, d), jnp.float32),
        scratch_types=[
            pltpu.VMEM((nchunk, E_CHUNK), jnp.int32),
        ] + [pltpu.VMEM((E_CHUNK, d), jnp.float32)] * NBUF + [
            pltpu.VMEM_SHARED((n_pad, d), jnp.float32),
        ] + [pltpu.SemaphoreType.DMA] * (NBUF + 1),
    )
    def segsum(h_hbm, dst_hbm, z_hbm, out_hbm, idx_v, *rest):
        rows = rest[:NBUF]
        acc_sh = rest[NBUF]
        lsem = rest[NBUF + 1:NBUF + 1 + NBUF]
        zsem = rest[2 * NBUF + 1]
        c = lax.axis_index("c")
        s = lax.axis_index("s")
        wid = s * NC + c
        zcp = pltpu.make_async_copy(
            z_hbm.at[pl.ds(s * rows_per_s, rows_per_s)],
            acc_sh.at[pl.ds(s * rows_per_s, rows_per_s)], zsem)
        zcp.start()
        pltpu.sync_copy(dst_hbm.at[wid], idx_v)
        base = wid * edges_per_w

        def start_load(i, b):
            blk = jnp.minimum(i, nchunk - 1)
            pltpu.make_async_copy(
                h_hbm.at[pl.ds(base + blk * E_CHUNK, E_CHUNK), :], rows[b],
                lsem[b]).start()

        for b in range(NBUF):
            start_load(jnp.int32(b), b)

        zcp.wait()
        plsc.subcore_barrier()

        def body(i, carry):
            for b in range(NBUF):
                @pl.when(i % NBUF == b)
                def _():
                    pltpu.make_async_copy(
                        h_hbm.at[pl.ds(0, E_CHUNK), :], rows[b],
                        lsem[b]).wait()
                    pltpu.sync_copy(rows[b], acc_sh.at[idx_v.at[i]], add=True)
                    start_load(i + NBUF, b)
            return carry

        lax.fori_loop(0, nchunk, body, 0)
        for b in range(NBUF):
            pltpu.make_async_copy(
                h_hbm.at[pl.ds(0, E_CHUNK), :], rows[b], lsem[b]).wait()
        plsc.subcore_barrier()
        pltpu.sync_copy(acc_sh.at[pl.ds(s * rows_per_s, rows_per_s)],
                        out_hbm.at[c, pl.ds(s * rows_per_s, rows_per_s)])

    return segsum(h, dst3, zeros)


def _mlp_tc(r, partials, W):
    n, d = r.shape
    blk = 2000

    def body(r_ref, p_ref, w_ref, o_ref):
        w = w_ref[...]
        msg = p_ref[0] + p_ref[1]
        acc = jnp.dot(r_ref[...], w[:d], preferred_element_type=jnp.float32)
        acc = acc + jnp.dot(msg, w[d:], preferred_element_type=jnp.float32)
        o_ref[...] = jnp.maximum(acc, 0.0)

    return pl.pallas_call(
        body,
        grid=(n // blk,),
        in_specs=[
            pl.BlockSpec((blk, d), lambda i: (i, 0)),
            pl.BlockSpec((2, blk, d), lambda i: (0, i, 0)),
            pl.BlockSpec((2 * d, d), lambda i: (0, 0)),
        ],
        out_specs=pl.BlockSpec((blk, d), lambda i: (i, 0)),
        out_shape=jax.ShapeDtypeStruct((n, d), jnp.float32),
    )(r, partials, W)


def kernel(r, h, nbrs, W):
    n_nodes, d = r.shape
    n_pad = ((n_nodes + NS * 8 - 1) // (NS * 8)) * (NS * 8)
    n_edges = h.shape[0]
    edges_per_w = n_edges // NW
    dst3 = nbrs[:, 0].astype(jnp.int32).reshape(
        NW, edges_per_w // E_CHUNK, E_CHUNK)
    zeros = jnp.zeros((n_pad, d), jnp.float32)
    partials = _segment_sum_sc(h, dst3, zeros, n_pad)
    return _mlp_tc(r, partials, W)


# speedup vs baseline: 2.3642x; 1.0041x over previous
import functools

import jax
import jax.numpy as jnp
from jax import lax
from jax.experimental import pallas as pl
from jax.experimental.pallas import tpu as pltpu
from jax.experimental.pallas import tpu_sc as plsc

NC = 2
NS = 16
NW = NC * NS
E_CHUNK = 80
NBUF = 3


def _segment_sum_sc(h, dst3, zeros, n_pad):
    n_edges, d = h.shape
    edges_per_w = n_edges // NW
    nchunk = edges_per_w // E_CHUNK
    rows_per_s = n_pad // NS

    mesh = plsc.VectorSubcoreMesh(core_axis_name="c", subcore_axis_name="s")

    @functools.partial(
        pl.kernel,
        mesh=mesh,
        out_type=jax.ShapeDtypeStruct((NC, n_pad, d), jnp.float32),
        scratch_types=[
            pltpu.VMEM((nchunk, E_CHUNK), jnp.int32),
        ] + [pltpu.VMEM((E_CHUNK, d), jnp.float32)] * NBUF + [
            pltpu.VMEM_SHARED((n_pad, d), jnp.float32),
        ] + [pltpu.SemaphoreType.DMA] * (NBUF + 1),
    )
    def segsum(h_hbm, dst_hbm, z_hbm, out_hbm, idx_v, *rest):
        rows = rest[:NBUF]
        acc_sh = rest[NBUF]
        lsem = rest[NBUF + 1:NBUF + 1 + NBUF]
        zsem = rest[2 * NBUF + 1]
        c = lax.axis_index("c")
        s = lax.axis_index("s")
        wid = s * NC + c
        zcp = pltpu.make_async_copy(
            z_hbm, acc_sh.at[pl.ds(s * rows_per_s, rows_per_s)], zsem)
        zcp.start()
        pltpu.sync_copy(dst_hbm.at[wid], idx_v)
        base = wid * edges_per_w

        def start_load(i, b):
            blk = jnp.minimum(i, nchunk - 1)
            pltpu.make_async_copy(
                h_hbm.at[pl.ds(base + blk * E_CHUNK, E_CHUNK), :], rows[b],
                lsem[b]).start()

        for b in range(NBUF):
            start_load(jnp.int32(b), b)

        zcp.wait()
        plsc.subcore_barrier()

        def body(i, carry):
            for b in range(NBUF):
                @pl.when(i % NBUF == b)
                def _():
                    pltpu.make_async_copy(
                        h_hbm.at[pl.ds(0, E_CHUNK), :], rows[b],
                        lsem[b]).wait()
                    pltpu.sync_copy(rows[b], acc_sh.at[idx_v.at[i]], add=True)
                    start_load(i + NBUF, b)
            return carry

        lax.fori_loop(0, nchunk, body, 0)
        for b in range(NBUF):
            pltpu.make_async_copy(
                h_hbm.at[pl.ds(0, E_CHUNK), :], rows[b], lsem[b]).wait()
        plsc.subcore_barrier()
        pltpu.sync_copy(acc_sh.at[pl.ds(s * rows_per_s, rows_per_s)],
                        out_hbm.at[c, pl.ds(s * rows_per_s, rows_per_s)])

    return segsum(h, dst3, zeros)


def _mlp_tc(r, partials, W):
    n, d = r.shape
    blk = 5000

    def body(r_ref, p_ref, w_ref, o_ref):
        w = w_ref[...]
        msg = p_ref[0] + p_ref[1]
        acc = jnp.dot(r_ref[...], w[:d], preferred_element_type=jnp.float32)
        acc = acc + jnp.dot(msg, w[d:], preferred_element_type=jnp.float32)
        o_ref[...] = jnp.maximum(acc, 0.0)

    return pl.pallas_call(
        body,
        grid=(n // blk,),
        in_specs=[
            pl.BlockSpec((blk, d), lambda i: (i, 0)),
            pl.BlockSpec((2, blk, d), lambda i: (0, i, 0)),
            pl.BlockSpec((2 * d, d), lambda i: (0, 0)),
        ],
        out_specs=pl.BlockSpec((blk, d), lambda i: (i, 0)),
        out_shape=jax.ShapeDtypeStruct((n, d), jnp.float32),
    )(r, partials, W)


def kernel(r, h, nbrs, W):
    n_nodes, d = r.shape
    n_pad = ((n_nodes + NS * 8 - 1) // (NS * 8)) * (NS * 8)
    n_edges = h.shape[0]
    edges_per_w = n_edges // NW
    dst3 = nbrs[:, 0].astype(jnp.int32).reshape(
        NW, edges_per_w // E_CHUNK, E_CHUNK)
    zeros = jnp.zeros((n_pad // NS, d), jnp.float32)
    partials = _segment_sum_sc(h, dst3, zeros, n_pad)
    return _mlp_tc(r, partials, W)
